# baseline (speedup 1.0000x reference)
import functools

import jax
import jax.numpy as jnp
from jax import lax
from jax.experimental import pallas as pl
from jax.experimental.pallas import tpu as pltpu
from jax.experimental.pallas import tpu_sc as plsc

_NUM_CODES = 1024
_ROWS = 18432
_BLK = 512
_GRID = _ROWS // _BLK

_COLOR_D = 2250
_POS_D = 450
_COLOR_DP = 2304
_POS_DP = 512

_NC = 2
_NS = 16
_NW = _NC * _NS
_BPW = _ROWS // _NW
_CH = 32
_NCHUNK = _BPW // _CH


def _stats_body(x_ref, idx_ref, ent_ref, acc_ref):
    i = pl.program_id(0)
    x = x_ref[...]
    m = jnp.max(x, axis=-1, keepdims=True)
    ids = lax.broadcasted_iota(jnp.int32, x.shape, 1)
    idx = jnp.min(jnp.where(x == m, ids, _NUM_CODES), axis=-1)
    idx_ref[...] = idx[None, None, :]

    e = jnp.exp(x - m)
    s = jnp.sum(e, axis=-1, keepdims=True)
    p = e / s

    @pl.when(i == 0)
    def _():
        acc_ref[...] = jnp.zeros_like(acc_ref)

    acc_ref[...] += jnp.sum(p, axis=0, keepdims=True)

    @pl.when(i == pl.num_programs(0) - 1)
    def _():
        avg = acc_ref[...] * (1.0 / _ROWS)
        ent_ref[0, 0] = -jnp.sum(avg * jnp.log(avg + 1e-8))


_stats_call = pl.pallas_call(
    _stats_body,
    grid=(_GRID,),
    in_specs=[pl.BlockSpec((_BLK, _NUM_CODES), lambda i: (i, 0))],
    out_specs=[
        pl.BlockSpec((1, 1, _BLK), lambda i: (i, 0, 0)),
        pl.BlockSpec(block_shape=(1, 1), index_map=lambda i: (0, 0),
                     memory_space=pltpu.SMEM),
    ],
    out_shape=[
        jax.ShapeDtypeStruct((_GRID, 1, _BLK), jnp.int32),
        jax.ShapeDtypeStruct((1, 1), jnp.float32),
    ],
    scratch_shapes=[pltpu.VMEM((1, _NUM_CODES), jnp.float32)],
)


def _gather_body(idx_hbm, color_hbm, pos_hbm, outc_hbm, outp_hbm,
                 idx_v, c_v, p_v, sem_c, sem_p):
    wid = lax.axis_index("s") * _NC + lax.axis_index("c")

    def chunk(c, carry):
        base = wid * _BPW + c * _CH
        pltpu.sync_copy(idx_hbm.at[pl.ds(base, _CH)], idx_v)
        cp_c = pltpu.async_copy(color_hbm.at[idx_v], c_v, sem_c)
        cp_p = pltpu.async_copy(pos_hbm.at[idx_v], p_v, sem_p)
        cp_c.wait()
        cp_p.wait()
        pltpu.sync_copy(c_v, outc_hbm.at[pl.ds(base, _CH)])
        pltpu.sync_copy(p_v, outp_hbm.at[pl.ds(base, _CH)])
        return carry

    lax.fori_loop(0, _NCHUNK, chunk, 0)


@functools.lru_cache(maxsize=1)
def _make_gather_call():
    return pl.kernel(
        _gather_body,
        out_type=[
            jax.ShapeDtypeStruct((_ROWS, _COLOR_DP), jnp.float32),
            jax.ShapeDtypeStruct((_ROWS, _POS_DP), jnp.float32),
        ],
        mesh=plsc.VectorSubcoreMesh(core_axis_name="c", subcore_axis_name="s"),
        scratch_types=[
            pltpu.VMEM((_CH,), jnp.int32),
            pltpu.VMEM((_CH, _COLOR_DP), jnp.float32),
            pltpu.VMEM((_CH, _POS_DP), jnp.float32),
            pltpu.SemaphoreType.DMA,
            pltpu.SemaphoreType.DMA,
        ],
    )


def kernel(logits, color_basis, pos_basis):
    b, k, c = logits.shape
    lf = logits.reshape(b * k, c)
    idx3d, ent = _stats_call(lf)
    idx_flat = idx3d.reshape(-1)
    color_p = jnp.pad(color_basis, ((0, 0), (0, _COLOR_DP - _COLOR_D)))
    pos_p = jnp.pad(pos_basis, ((0, 0), (0, _POS_DP - _POS_D)))
    colm, posm = _make_gather_call()(idx_flat, color_p, pos_p)
    colm = colm[:, :_COLOR_D]
    posm = posm[:, :_POS_D]
    return (
        colm.reshape(b, k, _COLOR_D),
        posm.reshape(b, k, _POS_D),
        idx_flat.reshape(b, k),
        ent[0, 0],
    )


# speedup vs baseline: 1.0130x; 1.0130x over previous
import functools

import jax
import jax.numpy as jnp
from jax import lax
from jax.experimental import pallas as pl
from jax.experimental.pallas import tpu as pltpu
from jax.experimental.pallas import tpu_sc as plsc

_NUM_CODES = 1024
_ROWS = 18432
_BLK = 512
_GRID = _ROWS // _BLK

_COLOR_D = 2250
_POS_D = 450
_COLOR_DP = 2304
_POS_DP = 512
_COLOR_MAIN = 2176
_POS_MAIN = 384
_COLOR_TAIL = _COLOR_D - _COLOR_MAIN
_POS_TAIL = _POS_D - _POS_MAIN

_NC = 2
_NS = 16
_NW = _NC * _NS
_BPW = _ROWS // _NW
_CH = 16
_NCHUNK = _BPW // _CH


def _stats_body(x_ref, idx_ref, ent_ref, acc_ref):
    i = pl.program_id(0)
    x = x_ref[...]
    m = jnp.max(x, axis=-1, keepdims=True)
    ids = lax.broadcasted_iota(jnp.int32, x.shape, 1)
    idx = jnp.min(jnp.where(x == m, ids, _NUM_CODES), axis=-1)
    idx_ref[...] = idx[None, None, :]

    e = jnp.exp(x - m)
    s = jnp.sum(e, axis=-1, keepdims=True)
    p = e / s

    @pl.when(i == 0)
    def _():
        acc_ref[...] = jnp.zeros_like(acc_ref)

    acc_ref[...] += jnp.sum(p, axis=0, keepdims=True)

    @pl.when(i == pl.num_programs(0) - 1)
    def _():
        avg = acc_ref[...] * (1.0 / _ROWS)
        ent_ref[0, 0] = -jnp.sum(avg * jnp.log(avg + 1e-8))


_stats_call = pl.pallas_call(
    _stats_body,
    grid=(_GRID,),
    in_specs=[pl.BlockSpec((_BLK, _NUM_CODES), lambda i: (i, 0))],
    out_specs=[
        pl.BlockSpec((1, 1, _BLK), lambda i: (i, 0, 0)),
        pl.BlockSpec(block_shape=(1, 1), index_map=lambda i: (0, 0),
                     memory_space=pltpu.SMEM),
    ],
    out_shape=[
        jax.ShapeDtypeStruct((_GRID, 1, _BLK), jnp.int32),
        jax.ShapeDtypeStruct((1, 1), jnp.float32),
    ],
    scratch_shapes=[pltpu.VMEM((1, _NUM_CODES), jnp.float32)],
)


def _compact_tail(src_v, slot, dst, col0, ncol):
    nfull = ncol // 16
    rem = ncol - nfull * 16
    lanes = lax.iota(jnp.int32, 16)
    mask = lanes < rem
    for r in range(16):
        for wv in range(nfull):
            dst[r, pl.ds(16 * wv, 16)] = src_v[slot, r, pl.ds(col0 + 16 * wv, 16)]
        v = src_v[slot, r, pl.ds(col0 + 16 * nfull, 16)]
        plsc.store_scatter(dst, [jnp.full((16,), r, jnp.int32),
                                 16 * nfull + lanes], v, mask=mask)


def _gather_body(idx_hbm, color_hbm, pos_hbm, outc_hbm, outp_hbm,
                 idx_v, c_v, p_v, ct_v, pt_v, sem0, sem1):
    wid = lax.axis_index("s") * _NC + lax.axis_index("c")
    row0 = wid * _BPW
    sems = (sem0, sem1)

    def start_chunk(c, slot):
        pltpu.sync_copy(idx_hbm.at[pl.ds(row0 + c * _CH, _CH)],
                        idx_v.at[slot])
        pltpu.async_copy(color_hbm.at[idx_v.at[slot]], c_v.at[slot],
                         sems[slot])
        pltpu.async_copy(pos_hbm.at[idx_v.at[slot]], p_v.at[slot],
                         sems[slot])

    def wait_chunk(slot):
        pltpu.make_async_copy(color_hbm.at[idx_v.at[slot]], c_v.at[slot],
                              sems[slot]).wait()
        pltpu.make_async_copy(pos_hbm.at[idx_v.at[slot]], p_v.at[slot],
                              sems[slot]).wait()

    def writeback(c, slot):
        base = row0 + c * _CH
        _compact_tail(c_v, slot, ct_v, _COLOR_MAIN, _COLOR_TAIL)
        _compact_tail(p_v, slot, pt_v, _POS_MAIN, _POS_TAIL)
        pltpu.sync_copy(c_v.at[slot, :, pl.ds(0, _COLOR_MAIN)],
                        outc_hbm.at[pl.ds(base, _CH), pl.ds(0, _COLOR_MAIN)])
        pltpu.sync_copy(ct_v,
                        outc_hbm.at[pl.ds(base, _CH),
                                    pl.ds(_COLOR_MAIN, _COLOR_TAIL)])
        pltpu.sync_copy(p_v.at[slot, :, pl.ds(0, _POS_MAIN)],
                        outp_hbm.at[pl.ds(base, _CH), pl.ds(0, _POS_MAIN)])
        pltpu.sync_copy(pt_v,
                        outp_hbm.at[pl.ds(base, _CH),
                                    pl.ds(_POS_MAIN, _POS_TAIL)])

    start_chunk(0, 0)

    def pair(g, carry):
        c0 = 2 * g
        wait_chunk(0)
        start_chunk(c0 + 1, 1)
        writeback(c0, 0)
        wait_chunk(1)

        @pl.when(c0 + 2 < _NCHUNK)
        def _():
            start_chunk(c0 + 2, 0)

        writeback(c0 + 1, 1)
        return carry

    lax.fori_loop(0, _NCHUNK // 2, pair, 0)


@functools.lru_cache(maxsize=1)
def _make_gather_call():
    return pl.kernel(
        _gather_body,
        out_type=[
            jax.ShapeDtypeStruct((_ROWS, _COLOR_D), jnp.float32),
            jax.ShapeDtypeStruct((_ROWS, _POS_D), jnp.float32),
        ],
        mesh=plsc.VectorSubcoreMesh(core_axis_name="c", subcore_axis_name="s"),
        scratch_types=[
            pltpu.VMEM((2, _CH), jnp.int32),
            pltpu.VMEM((2, _CH, _COLOR_DP), jnp.float32),
            pltpu.VMEM((2, _CH, _POS_DP), jnp.float32),
            pltpu.VMEM((_CH, _COLOR_TAIL), jnp.float32),
            pltpu.VMEM((_CH, _POS_TAIL), jnp.float32),
            pltpu.SemaphoreType.DMA,
            pltpu.SemaphoreType.DMA,
        ],
        compiler_params=pltpu.CompilerParams(needs_layout_passes=False),
    )


def kernel(logits, color_basis, pos_basis):
    b, k, c = logits.shape
    lf = logits.reshape(b * k, c)
    idx3d, ent = _stats_call(lf)
    idx_flat = idx3d.reshape(-1)
    color_p = jnp.pad(color_basis, ((0, 0), (0, _COLOR_DP - _COLOR_D)))
    pos_p = jnp.pad(pos_basis, ((0, 0), (0, _POS_DP - _POS_D)))
    colm, posm = _make_gather_call()(idx_flat, color_p, pos_p)
    return (
        colm.reshape(b, k, _COLOR_D),
        posm.reshape(b, k, _POS_D),
        idx_flat.reshape(b, k),
        ent[0, 0],
    )
